# baseline (speedup 1.0000x reference)
import jax
import jax.numpy as jnp
from jax import lax
from jax.experimental import pallas as pl

_N_SRC = 2048
_N_TGT = 768
_D = 1024
_TOP_N_SIM = 5
_RANK_K = 4
_TAU = 0.07
_TOP_RANKED = _N_TGT * 2 // 3
_NEG = -3.0
_BIG_I = 2 ** 30


def _msc_body(src_ref, lab_ref, tgt_ref, out_ref):
    src = src_ref[...]
    tgt = tgt_ref[...]
    lab = lab_ref[...]

    sn = jnp.maximum(jnp.sqrt(jnp.sum(src * src, axis=1, keepdims=True)), 1e-12)
    tn = jnp.maximum(jnp.sqrt(jnp.sum(tgt * tgt, axis=1, keepdims=True)), 1e-12)
    srcn = src / sn
    tgtn = tgt / tn

    sim = lax.dot_general(
        tgtn, srcn, (((1,), (1,)), ((), ())),
        preferred_element_type=jnp.float32,
    )

    col = lax.broadcasted_iota(jnp.int32, (_N_TGT, _N_SRC), 1)
    enc = col * 128 + lab

    simw = sim
    top_labs = []
    for _ in range(_TOP_N_SIM):
        vmax = jnp.max(simw, axis=1, keepdims=True)
        kmin = jnp.min(jnp.where(simw == vmax, enc, _BIG_I), axis=1, keepdims=True)
        top_labs.append(jnp.bitwise_and(kmin, 127))
        simw = jnp.where(enc == kmin, _NEG, simw)

    keys = []
    for a in range(_TOP_N_SIM):
        cnt = jnp.zeros((_N_TGT, 1), jnp.int32)
        for b in range(_TOP_N_SIM):
            cnt = cnt + (top_labs[a] == top_labs[b]).astype(jnp.int32)
        keys.append(cnt * 1048576 - top_labs[a])
    assigned = top_labs[0]
    best_key = keys[0]
    for a in range(1, _TOP_N_SIM):
        better = keys[a] > best_key
        assigned = jnp.where(better, top_labs[a], assigned)
        best_key = jnp.maximum(keys[a], best_key)

    like = lab == assigned

    ones_col = jnp.ones((_N_SRC, 1), jnp.float32)

    def top_k_sum(mask):
        w = jnp.where(mask, sim, _NEG)
        s = jnp.zeros((_N_TGT, 1), jnp.float32)
        rem = jnp.full((_N_TGT, 1), float(_RANK_K), jnp.float32)
        for _ in range(_RANK_K):
            vmax = jnp.max(w, axis=1, keepdims=True)
            hit = w == vmax
            c = lax.dot_general(hit.astype(jnp.float32), ones_col,
                                (((1,), (0,)), ((), ())),
                                preferred_element_type=jnp.float32)
            take = jnp.minimum(c, rem)
            s = s + jnp.where(vmax > -2.0, vmax * take, 0.0)
            rem = rem - take
            w = jnp.where(hit, _NEG, w)
        return s

    nln_sum = top_k_sum(like)
    nun_sum = top_k_sum(jnp.logical_not(like))
    scores = nln_sum / nun_sum

    m = jnp.max(sim, axis=1, keepdims=True)
    e = jnp.exp((sim - m) * (1.0 / _TAU))
    den = jnp.sum(e, axis=1, keepdims=True)
    num = jnp.sum(jnp.where(like, e, 0.0), axis=1, keepdims=True)
    lg = jnp.log(num / den + 1e-6)

    rI = lax.broadcasted_iota(jnp.int32, (_N_TGT, _N_TGT), 0)
    cI = lax.broadcasted_iota(jnp.int32, (_N_TGT, _N_TGT), 1)
    s_bc = jnp.broadcast_to(scores, (_N_TGT, _N_TGT))
    s_rv = jnp.sum(jnp.where(rI == cI, s_bc, 0.0), axis=0, keepdims=True)
    beats = (s_rv > scores) | ((s_rv == scores) & (cI < rI))
    rank = jnp.sum(beats.astype(jnp.int32), axis=1, keepdims=True)
    selected = rank < _TOP_RANKED

    total = jnp.sum(jnp.where(selected, lg, 0.0), axis=0, keepdims=True)
    out_ref[...] = -total / _TOP_RANKED


def kernel(source_features, source_labels, target_features):
    lab2d = source_labels.reshape(1, _N_SRC).astype(jnp.int32)
    out = pl.pallas_call(
        _msc_body,
        out_shape=jax.ShapeDtypeStruct((1, 1), jnp.float32),
    )(source_features, lab2d, target_features)
    return out[0, 0]


# speedup vs baseline: 1.0505x; 1.0505x over previous
import jax
import jax.numpy as jnp
from jax import lax
from jax.experimental import pallas as pl

_N_SRC = 2048
_N_TGT = 768
_D = 1024
_TOP_N_SIM = 5
_RANK_K = 4
_TAU = 0.07
_TOP_RANKED = _N_TGT * 2 // 3
_NEG = -3.0
_BIG_I = 2 ** 30


def _msc_body(src_ref, lab_ref, tgt_ref, out_ref):
    src = src_ref[...]
    tgt = tgt_ref[...]
    lab = lab_ref[...]

    sn = jnp.maximum(jnp.sqrt(jnp.sum(src * src, axis=1, keepdims=True)), 1e-12)
    tn = jnp.maximum(jnp.sqrt(jnp.sum(tgt * tgt, axis=1, keepdims=True)), 1e-12)
    srcn = src / sn
    tgtn = tgt / tn

    sim = lax.dot_general(
        tgtn, srcn, (((1,), (1,)), ((), ())),
        preferred_element_type=jnp.float32,
    )

    col1 = lax.broadcasted_iota(jnp.int32, (1, _N_SRC), 1)
    enc = col1 * 128 + lab

    simw = sim
    top_labs = []
    m = None
    for _ in range(_TOP_N_SIM):
        vmax = jnp.max(simw, axis=1, keepdims=True)
        if m is None:
            m = vmax
        kmin = jnp.min(jnp.where(simw == vmax, enc, _BIG_I), axis=1, keepdims=True)
        top_labs.append(jnp.bitwise_and(kmin, 127))
        simw = jnp.where(enc == kmin, _NEG, simw)

    keys = []
    for a in range(_TOP_N_SIM):
        cnt = jnp.zeros((_N_TGT, 1), jnp.int32)
        for b in range(_TOP_N_SIM):
            cnt = cnt + (top_labs[a] == top_labs[b]).astype(jnp.int32)
        keys.append(cnt * 1048576 - top_labs[a])
    assigned = top_labs[0]
    best_key = keys[0]
    for a in range(1, _TOP_N_SIM):
        better = keys[a] > best_key
        assigned = jnp.where(better, top_labs[a], assigned)
        best_key = jnp.maximum(keys[a], best_key)

    like = lab == assigned

    def top_k_sum(mask):
        w = jnp.where(mask, sim, _NEG)
        s = jnp.zeros((_N_TGT, 1), jnp.float32)
        rem = jnp.full((_N_TGT, 1), float(_RANK_K), jnp.float32)
        for _ in range(_RANK_K):
            vmax = jnp.max(w, axis=1, keepdims=True)
            hit = w == vmax
            c = jnp.sum(hit.astype(jnp.float32), axis=1, keepdims=True)
            take = jnp.minimum(c, rem)
            s = s + jnp.where(vmax > -2.0, vmax * take, 0.0)
            rem = rem - take
            w = jnp.where(hit, _NEG, w)
        return s

    nln_sum = top_k_sum(like)
    nun_sum = top_k_sum(jnp.logical_not(like))
    scores = nln_sum / nun_sum

    e = jnp.exp((sim - m) * (1.0 / _TAU))
    den = jnp.sum(e, axis=1, keepdims=True)
    num = jnp.sum(jnp.where(like, e, 0.0), axis=1, keepdims=True)
    lg = jnp.log(num / den + 1e-6)

    rI = lax.broadcasted_iota(jnp.int32, (_N_TGT, _N_TGT), 0)
    cI = lax.broadcasted_iota(jnp.int32, (_N_TGT, _N_TGT), 1)
    s_bc = jnp.broadcast_to(scores, (_N_TGT, _N_TGT))
    s_rv = jnp.sum(jnp.where(rI == cI, s_bc, 0.0), axis=0, keepdims=True)
    beats = (s_rv > scores) | ((s_rv == scores) & (cI < rI))
    rank = jnp.sum(beats.astype(jnp.int32), axis=1, keepdims=True)
    selected = rank < _TOP_RANKED

    total = jnp.sum(jnp.where(selected, lg, 0.0), axis=0, keepdims=True)
    out_ref[...] = -total / _TOP_RANKED


def kernel(source_features, source_labels, target_features):
    lab2d = source_labels.reshape(1, _N_SRC).astype(jnp.int32)
    out = pl.pallas_call(
        _msc_body,
        out_shape=jax.ShapeDtypeStruct((1, 1), jnp.float32),
    )(source_features, lab2d, target_features)
    return out[0, 0]
